# baseline (speedup 1.0000x reference)
import functools

import jax
import jax.numpy as jnp
from jax import lax
from jax.experimental import pallas as pl
from jax.experimental.pallas import tpu as pltpu
from jax.experimental.pallas import tpu_sc as plsc

N = 50000
E = 800000
F = 64
FE = 16
G = 64
H = 256
FG = 16
IN_DIM = F + FE + FG

NP = 50048
ROWS_PER_TILE = NP // 16

NVEC = E // 128
NW = 32
NG_FULL = NVEC // 8
NG_BASE = NG_FULL // NW
NG_EXTRA = NG_FULL - NG_BASE * NW
TAIL_V = NG_FULL * 8

BN = 2000
NB = N // BN


def _sc_body(ei_ref, attr_ref, sums_out, cnts_out,
             sums_sh, cnts_sh, idx_v, attr_v, ones_v, dma_sem, sc_sem):
    cid = lax.axis_index("c")
    sid = lax.axis_index("s")
    w = sid * 2 + cid

    def _fill_z(i, carry):
        attr_v[i, :] = jnp.zeros((16,), jnp.float32)
        return carry
    lax.fori_loop(0, 1024, _fill_z, None)

    def _fill_o(i, carry):
        ones_v[i, :] = jnp.ones((16,), jnp.float32)
        return carry
    lax.fori_loop(0, 128, _fill_o, None)

    base = sid * ROWS_PER_TILE
    for q in range(3):
        pltpu.sync_copy(attr_v, sums_sh.at[pl.ds(base + q * 1024, 1024)])
        pltpu.sync_copy(attr_v, cnts_sh.at[pl.ds(base + q * 1024, 1024)])
    pltpu.sync_copy(attr_v.at[pl.ds(0, 56)],
                    sums_sh.at[pl.ds(base + 3072, 56)])
    pltpu.sync_copy(attr_v.at[pl.ds(0, 56)],
                    cnts_sh.at[pl.ds(base + 3072, 56)])
    plsc.subcore_barrier()

    gbase = w * NG_BASE + jnp.minimum(w, NG_EXTRA)
    ngroups = NG_BASE + jnp.where(w < NG_EXTRA, 1, 0)

    def _group(g2, carry):
        v0 = (gbase + g2) * 8
        fetches = [pltpu.async_copy(ei_ref.at[1, pl.ds(v0, 8)], idx_v,
                                    dma_sem)]
        fetches.append(pltpu.async_copy(
            attr_ref.at[pl.ds(v0 * 128, 8 * 128)], attr_v, dma_sem))
        for c in fetches:
            c.wait()
        scatters = []
        for j in range(8):
            scatters.append(pltpu.async_copy(
                attr_v.at[pl.ds(j * 128, 128)],
                sums_sh.at[idx_v.at[j]], sc_sem, add=True))
            scatters.append(pltpu.async_copy(
                ones_v, cnts_sh.at[idx_v.at[j]], sc_sem, add=True))
        for c in scatters:
            c.wait()
        return carry
    lax.fori_loop(0, ngroups, _group, None)

    @pl.when(w == NW - 1)
    def _tail():
        f1 = pltpu.async_copy(ei_ref.at[1, pl.ds(TAIL_V, 2)],
                              idx_v.at[pl.ds(0, 2)], dma_sem)
        f2 = pltpu.async_copy(attr_ref.at[pl.ds(TAIL_V * 128, 2 * 128)],
                              attr_v.at[pl.ds(0, 2 * 128)], dma_sem)
        f1.wait()
        f2.wait()
        scatters = []
        for j in range(2):
            scatters.append(pltpu.async_copy(
                attr_v.at[pl.ds(j * 128, 128)],
                sums_sh.at[idx_v.at[j]], sc_sem, add=True))
            scatters.append(pltpu.async_copy(
                ones_v, cnts_sh.at[idx_v.at[j]], sc_sem, add=True))
        for c in scatters:
            c.wait()

    plsc.subcore_barrier()

    for acc, dst in ((sums_sh, sums_out), (cnts_sh, cnts_out)):
        for q in range(3):
            pltpu.sync_copy(acc.at[pl.ds(base + q * 1024, 1024)], attr_v)
            pltpu.sync_copy(attr_v, dst.at[cid, pl.ds(base + q * 1024, 1024)])
        pltpu.sync_copy(acc.at[pl.ds(base + 3072, 56)],
                        attr_v.at[pl.ds(0, 56)])
        pltpu.sync_copy(attr_v.at[pl.ds(0, 56)],
                        dst.at[cid, pl.ds(base + 3072, 56)])


@functools.cache
def _make_sc_segment():
    return pl.kernel(
        _sc_body,
        out_type=(jax.ShapeDtypeStruct((2, NP, 16), jnp.float32),
                  jax.ShapeDtypeStruct((2, NP, 16), jnp.float32)),
        mesh=plsc.VectorSubcoreMesh(core_axis_name="c", subcore_axis_name="s",
                                    num_cores=2, num_subcores=16),
        scratch_types=[
            pltpu.VMEM_SHARED((NP, 16), jnp.float32),
            pltpu.VMEM_SHARED((NP, 16), jnp.float32),
            pltpu.VMEM((8, 128), jnp.int32),
            pltpu.VMEM((8 * 128, 16), jnp.float32),
            pltpu.VMEM((128, 16), jnp.float32),
            pltpu.SemaphoreType.DMA,
            pltpu.SemaphoreType.DMA,
        ],
        compiler_params=pltpu.CompilerParams(use_tc_tiling_on_sc=False),
    )


TRB = E // 50


def _att_body(at_ref, out_ref):
    eye = (lax.broadcasted_iota(jnp.int32, (FE, FE), 0)
           == lax.broadcasted_iota(jnp.int32, (FE, FE), 1)).astype(jnp.float32)
    out_ref[...] = lax.dot_general(
        at_ref[...], eye, (((0,), (0,)), ((), ())),
        precision=lax.Precision.HIGHEST,
        preferred_element_type=jnp.float32)


def _attr_rowmajor(edge_attr):
    attr_t = edge_attr.T
    return pl.pallas_call(
        _att_body,
        grid=(E // TRB,),
        in_specs=[pl.BlockSpec((FE, TRB), lambda i: (0, i))],
        out_specs=pl.BlockSpec((TRB, FE), lambda i: (i, 0)),
        out_shape=jax.ShapeDtypeStruct((E, FE), jnp.float32),
    )(attr_t)


def _build_out(x_ref, sp_ref, cp_ref, u_ref, b_ref):
    x = x_ref[...]
    e = (sp_ref[0] + sp_ref[1]) / jnp.maximum(cp_ref[0] + cp_ref[1], 1.0)
    bidx = b_ref[0, 0, :]
    oh = (bidx[:, None] == lax.broadcasted_iota(jnp.int32, (BN, G), 1)
          ).astype(jnp.float32)
    ub = jnp.dot(oh, u_ref[...], preferred_element_type=jnp.float32)
    return jnp.concatenate([x, e, ub], axis=1)


def _moments_body(x_ref, sp_ref, cp_ref, u_ref, b_ref, W1_ref, sh_ref, sh2_ref):
    i = pl.program_id(0)

    @pl.when(i == 0)
    def _init():
        sh_ref[...] = jnp.zeros_like(sh_ref)
        sh2_ref[...] = jnp.zeros_like(sh2_ref)

    out = _build_out(x_ref, sp_ref, cp_ref, u_ref, b_ref)
    g = jnp.dot(out, W1_ref[...], preferred_element_type=jnp.float32)
    sh_ref[...] += jnp.broadcast_to(jnp.sum(g, axis=0, keepdims=True), (8, H))
    sh2_ref[...] += jnp.broadcast_to(
        jnp.sum(g * g, axis=0, keepdims=True), (8, H))


def _fold_body(sh_ref, sh2_ref, gm_ref, bt_ref, W1_ref, W1f_ref, b1f_ref):
    inv_n = jnp.float32(1.0 / N)
    mean_g = sh_ref[0:1, :] * inv_n
    var = sh2_ref[0:1, :] * inv_n - mean_g * mean_g
    scale = gm_ref[0:1, :] * lax.rsqrt(var + 1e-5)
    W1f_ref[...] = W1_ref[...] * scale
    b1f_ref[...] = jnp.broadcast_to(bt_ref[0:1, :] - mean_g * scale, (8, H))


def _final_body(x_ref, sp_ref, cp_ref, u_ref, b_ref, W1f_ref, b1f_ref,
                W2_ref, b2_ref, y_ref):
    out = _build_out(x_ref, sp_ref, cp_ref, u_ref, b_ref)
    g = jnp.dot(out, W1f_ref[...], preferred_element_type=jnp.float32)
    h = jnp.maximum(g + b1f_ref[0:1, :], 0.0)
    y_ref[...] = (jnp.dot(h, W2_ref[...], preferred_element_type=jnp.float32)
                  + b2_ref[0:1, :])


_node_specs = [
    pl.BlockSpec((BN, F), lambda i: (i, 0)),
    pl.BlockSpec((2, BN, FE), lambda i: (0, i, 0)),
    pl.BlockSpec((2, BN, FE), lambda i: (0, i, 0)),
    pl.BlockSpec((G, FG), lambda i: (0, 0)),
    pl.BlockSpec((1, 1, BN), lambda i: (i, 0, 0)),
]


def kernel(x, edge_index, edge_attr, u, batch, W1, b1, gamma, beta, W2, b2):
    del b1
    ei3 = edge_index.astype(jnp.int32).reshape(2, NVEC, 128)
    attr_rm = _attr_rowmajor(edge_attr)

    sp, cp = _make_sc_segment()(ei3, attr_rm)

    batch3 = batch.astype(jnp.int32).reshape(NB, 1, BN)
    gammar = jnp.broadcast_to(gamma.reshape(1, H), (8, H))
    betar = jnp.broadcast_to(beta.reshape(1, H), (8, H))
    b2r = jnp.broadcast_to(b2.reshape(1, F), (8, F))

    sh, sh2 = pl.pallas_call(
        _moments_body,
        grid=(NB,),
        in_specs=_node_specs + [pl.BlockSpec((IN_DIM, H), lambda i: (0, 0))],
        out_specs=(pl.BlockSpec((8, H), lambda i: (0, 0)),
                   pl.BlockSpec((8, H), lambda i: (0, 0))),
        out_shape=(jax.ShapeDtypeStruct((8, H), jnp.float32),
                   jax.ShapeDtypeStruct((8, H), jnp.float32)),
    )(x, sp, cp, u, batch3, W1)

    W1f, b1f = pl.pallas_call(
        _fold_body,
        out_shape=(jax.ShapeDtypeStruct((IN_DIM, H), jnp.float32),
                   jax.ShapeDtypeStruct((8, H), jnp.float32)),
    )(sh, sh2, gammar, betar, W1)

    y = pl.pallas_call(
        _final_body,
        grid=(NB,),
        in_specs=_node_specs + [
            pl.BlockSpec((IN_DIM, H), lambda i: (0, 0)),
            pl.BlockSpec((8, H), lambda i: (0, 0)),
            pl.BlockSpec((H, F), lambda i: (0, 0)),
            pl.BlockSpec((8, F), lambda i: (0, 0)),
        ],
        out_specs=pl.BlockSpec((BN, F), lambda i: (i, 0)),
        out_shape=jax.ShapeDtypeStruct((N, F), jnp.float32),
    )(x, sp, cp, u, batch3, W1f, b1f, W2, b2r)

    return y


# speedup vs baseline: 1.4533x; 1.4533x over previous
import functools

import jax
import jax.numpy as jnp
from jax import lax
from jax.experimental import pallas as pl
from jax.experimental.pallas import tpu as pltpu
from jax.experimental.pallas import tpu_sc as plsc

N = 50000
E = 800000
F = 64
FE = 16
G = 64
H = 256
FG = 16
IN_DIM = F + FE + FG

NP = 50048
ROWS_PER_TILE = NP // 16

NVEC = E // 128
NW = 32
NG_FULL = NVEC // 8
NG_BASE = NG_FULL // NW
NG_EXTRA = NG_FULL - NG_BASE * NW
TAIL_V = NG_FULL * 8

BN = 2000
NB = N // BN


def _sc_body(ei_ref, attr_ref, sums_out, cnts_out,
             sums_sh, cnts_sh, idx_v, attr_v, ones_v, dma_sem, sc_sem):
    cid = lax.axis_index("c")
    sid = lax.axis_index("s")
    w = sid * 2 + cid

    def _fill_z(i, carry):
        attr_v[i, :] = jnp.zeros((16,), jnp.float32)
        return carry
    lax.fori_loop(0, 1024, _fill_z, None)

    def _fill_o(i, carry):
        ones_v[i, :] = jnp.ones((16,), jnp.float32)
        return carry
    lax.fori_loop(0, 128, _fill_o, None)

    base = sid * ROWS_PER_TILE
    for q in range(3):
        pltpu.sync_copy(attr_v, sums_sh.at[pl.ds(base + q * 1024, 1024)])
        pltpu.sync_copy(attr_v, cnts_sh.at[pl.ds(base + q * 1024, 1024)])
    pltpu.sync_copy(attr_v.at[pl.ds(0, 56)],
                    sums_sh.at[pl.ds(base + 3072, 56)])
    pltpu.sync_copy(attr_v.at[pl.ds(0, 56)],
                    cnts_sh.at[pl.ds(base + 3072, 56)])
    plsc.subcore_barrier()

    gbase = w * NG_BASE + jnp.minimum(w, NG_EXTRA)
    ngroups = NG_BASE + jnp.where(w < NG_EXTRA, 1, 0)

    def _group(g2, carry):
        v0 = (gbase + g2) * 8
        fetches = [pltpu.async_copy(ei_ref.at[1, pl.ds(v0, 8)], idx_v,
                                    dma_sem)]
        fetches.append(pltpu.async_copy(
            attr_ref.at[pl.ds(v0 * 128, 8 * 128)], attr_v, dma_sem))
        for c in fetches:
            c.wait()
        scatters = []
        for j in range(8):
            scatters.append(pltpu.async_copy(
                attr_v.at[pl.ds(j * 128, 128)],
                sums_sh.at[idx_v.at[j]], sc_sem, add=True))
            scatters.append(pltpu.async_copy(
                ones_v, cnts_sh.at[idx_v.at[j]], sc_sem, add=True))
        for c in scatters:
            c.wait()
        return carry
    lax.fori_loop(0, ngroups, _group, None)

    @pl.when(w == NW - 1)
    def _tail():
        f1 = pltpu.async_copy(ei_ref.at[1, pl.ds(TAIL_V, 2)],
                              idx_v.at[pl.ds(0, 2)], dma_sem)
        f2 = pltpu.async_copy(attr_ref.at[pl.ds(TAIL_V * 128, 2 * 128)],
                              attr_v.at[pl.ds(0, 2 * 128)], dma_sem)
        f1.wait()
        f2.wait()
        scatters = []
        for j in range(2):
            scatters.append(pltpu.async_copy(
                attr_v.at[pl.ds(j * 128, 128)],
                sums_sh.at[idx_v.at[j]], sc_sem, add=True))
            scatters.append(pltpu.async_copy(
                ones_v, cnts_sh.at[idx_v.at[j]], sc_sem, add=True))
        for c in scatters:
            c.wait()

    plsc.subcore_barrier()

    for acc, dst in ((sums_sh, sums_out), (cnts_sh, cnts_out)):
        for q in range(3):
            pltpu.sync_copy(acc.at[pl.ds(base + q * 1024, 1024)], attr_v)
            pltpu.sync_copy(attr_v, dst.at[cid, pl.ds(base + q * 1024, 1024)])
        pltpu.sync_copy(acc.at[pl.ds(base + 3072, 56)],
                        attr_v.at[pl.ds(0, 56)])
        pltpu.sync_copy(attr_v.at[pl.ds(0, 56)],
                        dst.at[cid, pl.ds(base + 3072, 56)])


@functools.cache
def _make_sc_segment():
    return pl.kernel(
        _sc_body,
        out_type=(jax.ShapeDtypeStruct((2, NP, 16), jnp.float32),
                  jax.ShapeDtypeStruct((2, NP, 16), jnp.float32)),
        mesh=plsc.VectorSubcoreMesh(core_axis_name="c", subcore_axis_name="s",
                                    num_cores=2, num_subcores=16),
        scratch_types=[
            pltpu.VMEM_SHARED((NP, 16), jnp.float32),
            pltpu.VMEM_SHARED((NP, 16), jnp.float32),
            pltpu.VMEM((8, 128), jnp.int32),
            pltpu.VMEM((8 * 128, 16), jnp.float32),
            pltpu.VMEM((128, 16), jnp.float32),
            pltpu.SemaphoreType.DMA,
            pltpu.SemaphoreType.DMA,
        ],
        compiler_params=pltpu.CompilerParams(use_tc_tiling_on_sc=False),
    )


def _build_out(x_ref, sp_ref, cp_ref, u_ref, b_ref):
    x = x_ref[...]
    e = (sp_ref[0] + sp_ref[1]) / jnp.maximum(cp_ref[0] + cp_ref[1], 1.0)
    bidx = b_ref[0, 0, :]
    oh = (bidx[:, None] == lax.broadcasted_iota(jnp.int32, (BN, G), 1)
          ).astype(jnp.float32)
    ub = jnp.dot(oh, u_ref[...], preferred_element_type=jnp.float32)
    return jnp.concatenate([x, e, ub], axis=1)


def _moments_body(x_ref, sp_ref, cp_ref, u_ref, b_ref, W1_ref, sh_ref, sh2_ref):
    i = pl.program_id(0)

    @pl.when(i == 0)
    def _init():
        sh_ref[...] = jnp.zeros_like(sh_ref)
        sh2_ref[...] = jnp.zeros_like(sh2_ref)

    out = _build_out(x_ref, sp_ref, cp_ref, u_ref, b_ref)
    g = jnp.dot(out, W1_ref[...], preferred_element_type=jnp.float32)
    sh_ref[...] += jnp.broadcast_to(jnp.sum(g, axis=0, keepdims=True), (8, H))
    sh2_ref[...] += jnp.broadcast_to(
        jnp.sum(g * g, axis=0, keepdims=True), (8, H))


def _fold_body(sh_ref, sh2_ref, gm_ref, bt_ref, W1_ref, W1f_ref, b1f_ref):
    inv_n = jnp.float32(1.0 / N)
    mean_g = sh_ref[0:1, :] * inv_n
    var = sh2_ref[0:1, :] * inv_n - mean_g * mean_g
    scale = gm_ref[0:1, :] * lax.rsqrt(var + 1e-5)
    W1f_ref[...] = W1_ref[...] * scale
    b1f_ref[...] = jnp.broadcast_to(bt_ref[0:1, :] - mean_g * scale, (8, H))


def _final_body(x_ref, sp_ref, cp_ref, u_ref, b_ref, W1f_ref, b1f_ref,
                W2_ref, b2_ref, y_ref):
    out = _build_out(x_ref, sp_ref, cp_ref, u_ref, b_ref)
    g = jnp.dot(out, W1f_ref[...], preferred_element_type=jnp.float32)
    h = jnp.maximum(g + b1f_ref[0:1, :], 0.0)
    y_ref[...] = (jnp.dot(h, W2_ref[...], preferred_element_type=jnp.float32)
                  + b2_ref[0:1, :])


_node_specs = [
    pl.BlockSpec((BN, F), lambda i: (i, 0)),
    pl.BlockSpec((2, BN, FE), lambda i: (0, i, 0)),
    pl.BlockSpec((2, BN, FE), lambda i: (0, i, 0)),
    pl.BlockSpec((G, FG), lambda i: (0, 0)),
    pl.BlockSpec((1, 1, BN), lambda i: (i, 0, 0)),
]


def kernel(x, edge_index, edge_attr, u, batch, W1, b1, gamma, beta, W2, b2):
    del b1
    ei3 = edge_index.astype(jnp.int32).reshape(2, NVEC, 128)

    sp, cp = _make_sc_segment()(ei3, edge_attr)

    batch3 = batch.astype(jnp.int32).reshape(NB, 1, BN)
    gammar = jnp.broadcast_to(gamma.reshape(1, H), (8, H))
    betar = jnp.broadcast_to(beta.reshape(1, H), (8, H))
    b2r = jnp.broadcast_to(b2.reshape(1, F), (8, F))

    sh, sh2 = pl.pallas_call(
        _moments_body,
        grid=(NB,),
        in_specs=_node_specs + [pl.BlockSpec((IN_DIM, H), lambda i: (0, 0))],
        out_specs=(pl.BlockSpec((8, H), lambda i: (0, 0)),
                   pl.BlockSpec((8, H), lambda i: (0, 0))),
        out_shape=(jax.ShapeDtypeStruct((8, H), jnp.float32),
                   jax.ShapeDtypeStruct((8, H), jnp.float32)),
    )(x, sp, cp, u, batch3, W1)

    W1f, b1f = pl.pallas_call(
        _fold_body,
        out_shape=(jax.ShapeDtypeStruct((IN_DIM, H), jnp.float32),
                   jax.ShapeDtypeStruct((8, H), jnp.float32)),
    )(sh, sh2, gammar, betar, W1)

    y = pl.pallas_call(
        _final_body,
        grid=(NB,),
        in_specs=_node_specs + [
            pl.BlockSpec((IN_DIM, H), lambda i: (0, 0)),
            pl.BlockSpec((8, H), lambda i: (0, 0)),
            pl.BlockSpec((H, F), lambda i: (0, 0)),
            pl.BlockSpec((8, F), lambda i: (0, 0)),
        ],
        out_specs=pl.BlockSpec((BN, F), lambda i: (i, 0)),
        out_shape=jax.ShapeDtypeStruct((N, F), jnp.float32),
    )(x, sp, cp, u, batch3, W1f, b1f, W2, b2r)

    return y


# speedup vs baseline: 1.4908x; 1.0258x over previous
import functools

import jax
import jax.numpy as jnp
from jax import lax
from jax.experimental import pallas as pl
from jax.experimental.pallas import tpu as pltpu
from jax.experimental.pallas import tpu_sc as plsc

N = 50000
E = 800000
F = 64
FE = 16
G = 64
H = 256
FG = 16
IN_DIM = F + FE + FG

NP = 50048
ROWS_PER_TILE = NP // 16

NVEC = E // 128
NW = 32
NG_FULL = NVEC // 8
NG_BASE = NG_FULL // NW
NG_EXTRA = NG_FULL - NG_BASE * NW
TAIL_V = NG_FULL * 8

BN = 2000
NB = N // BN


def _sc_body(ei_ref, attr_ref, sums_out, cnts_out,
             sums_sh, cnts_sh, idx_v, attr_v, ones_v, dma_sem, sc_sem):
    cid = lax.axis_index("c")
    sid = lax.axis_index("s")
    w = sid * 2 + cid

    def _fill_z(i, carry):
        attr_v[i, :] = jnp.zeros((16,), jnp.float32)
        return carry
    lax.fori_loop(0, 1024, _fill_z, None)

    def _fill_o(i, carry):
        ones_v[i, :] = jnp.ones((16,), jnp.float32)
        return carry
    lax.fori_loop(0, 128, _fill_o, None)

    base = sid * ROWS_PER_TILE
    for q in range(3):
        pltpu.sync_copy(attr_v, sums_sh.at[pl.ds(base + q * 1024, 1024)])
        pltpu.sync_copy(attr_v, cnts_sh.at[pl.ds(base + q * 1024, 1024)])
    pltpu.sync_copy(attr_v.at[pl.ds(0, 56)],
                    sums_sh.at[pl.ds(base + 3072, 56)])
    pltpu.sync_copy(attr_v.at[pl.ds(0, 56)],
                    cnts_sh.at[pl.ds(base + 3072, 56)])
    plsc.subcore_barrier()

    gbase = w * NG_BASE + jnp.minimum(w, NG_EXTRA)
    ngroups = NG_BASE + jnp.where(w < NG_EXTRA, 1, 0)

    def _group(g2, carry):
        v0 = (gbase + g2) * 8
        fetches = [pltpu.async_copy(ei_ref.at[1, pl.ds(v0, 8)], idx_v,
                                    dma_sem)]
        fetches.append(pltpu.async_copy(
            attr_ref.at[pl.ds(v0 * 128, 8 * 128)], attr_v, dma_sem))
        for c in fetches:
            c.wait()
        scatters = []
        for j in range(8):
            scatters.append(pltpu.async_copy(
                attr_v.at[pl.ds(j * 128, 128)],
                sums_sh.at[idx_v.at[j]], sc_sem, add=True))
            scatters.append(pltpu.async_copy(
                ones_v, cnts_sh.at[idx_v.at[j]], sc_sem, add=True))
        for c in scatters:
            c.wait()
        return carry
    lax.fori_loop(0, ngroups, _group, None)

    @pl.when(w == NW - 1)
    def _tail():
        f1 = pltpu.async_copy(ei_ref.at[1, pl.ds(TAIL_V, 2)],
                              idx_v.at[pl.ds(0, 2)], dma_sem)
        f2 = pltpu.async_copy(attr_ref.at[pl.ds(TAIL_V * 128, 2 * 128)],
                              attr_v.at[pl.ds(0, 2 * 128)], dma_sem)
        f1.wait()
        f2.wait()
        scatters = []
        for j in range(2):
            scatters.append(pltpu.async_copy(
                attr_v.at[pl.ds(j * 128, 128)],
                sums_sh.at[idx_v.at[j]], sc_sem, add=True))
            scatters.append(pltpu.async_copy(
                ones_v, cnts_sh.at[idx_v.at[j]], sc_sem, add=True))
        for c in scatters:
            c.wait()

    plsc.subcore_barrier()

    for acc, dst in ((sums_sh, sums_out), (cnts_sh, cnts_out)):
        for q in range(3):
            pltpu.sync_copy(acc.at[pl.ds(base + q * 1024, 1024)], attr_v)
            pltpu.sync_copy(attr_v, dst.at[cid, pl.ds(base + q * 1024, 1024)])
        pltpu.sync_copy(acc.at[pl.ds(base + 3072, 56)],
                        attr_v.at[pl.ds(0, 56)])
        pltpu.sync_copy(attr_v.at[pl.ds(0, 56)],
                        dst.at[cid, pl.ds(base + 3072, 56)])


@functools.cache
def _make_sc_segment():
    return pl.kernel(
        _sc_body,
        out_type=(jax.ShapeDtypeStruct((2, NP, 16), jnp.float32),
                  jax.ShapeDtypeStruct((2, NP, 16), jnp.float32)),
        mesh=plsc.VectorSubcoreMesh(core_axis_name="c", subcore_axis_name="s",
                                    num_cores=2, num_subcores=16),
        scratch_types=[
            pltpu.VMEM_SHARED((NP, 16), jnp.float32),
            pltpu.VMEM_SHARED((NP, 16), jnp.float32),
            pltpu.VMEM((8, 128), jnp.int32),
            pltpu.VMEM((8 * 128, 16), jnp.float32),
            pltpu.VMEM((128, 16), jnp.float32),
            pltpu.SemaphoreType.DMA,
            pltpu.SemaphoreType.DMA,
        ],
        compiler_params=pltpu.CompilerParams(use_tc_tiling_on_sc=False),
    )


def _build_out(x_ref, sp_ref, cp_ref, u_ref, b_ref):
    x = x_ref[...]
    e = (sp_ref[0] + sp_ref[1]) / jnp.maximum(cp_ref[0] + cp_ref[1], 1.0)
    bidx = b_ref[0, 0, :]
    oh = (bidx[:, None] == lax.broadcasted_iota(jnp.int32, (BN, G), 1)
          ).astype(jnp.float32)
    ub = jnp.dot(oh, u_ref[...], preferred_element_type=jnp.float32)
    return jnp.concatenate([x, e, ub], axis=1)


def _moments_body(x_ref, sp_ref, cp_ref, u_ref, b_ref, W1_ref,
                  sh_ref, sh2_ref, outc_ref):
    i = pl.program_id(0)

    @pl.when(i == 0)
    def _init():
        sh_ref[...] = jnp.zeros_like(sh_ref)
        sh2_ref[...] = jnp.zeros_like(sh2_ref)

    out = _build_out(x_ref, sp_ref, cp_ref, u_ref, b_ref)
    outc_ref[...] = out
    g = jnp.dot(out, W1_ref[...], preferred_element_type=jnp.float32)
    sh_ref[...] += jnp.broadcast_to(jnp.sum(g, axis=0, keepdims=True), (8, H))
    sh2_ref[...] += jnp.broadcast_to(
        jnp.sum(g * g, axis=0, keepdims=True), (8, H))


def _fold_body(sh_ref, sh2_ref, gm_ref, bt_ref, W1_ref, W1f_ref, b1f_ref):
    inv_n = jnp.float32(1.0 / N)
    mean_g = sh_ref[0:1, :] * inv_n
    var = sh2_ref[0:1, :] * inv_n - mean_g * mean_g
    scale = gm_ref[0:1, :] * lax.rsqrt(var + 1e-5)
    W1f_ref[...] = W1_ref[...] * scale
    b1f_ref[...] = jnp.broadcast_to(bt_ref[0:1, :] - mean_g * scale, (8, H))


def _final_body(outc_ref, W1f_ref, b1f_ref, W2_ref, b2_ref, y_ref):
    out = outc_ref[...]
    g = jnp.dot(out, W1f_ref[...], preferred_element_type=jnp.float32)
    h = jnp.maximum(g + b1f_ref[0:1, :], 0.0)
    y_ref[...] = (jnp.dot(h, W2_ref[...], preferred_element_type=jnp.float32)
                  + b2_ref[0:1, :])


_node_specs = [
    pl.BlockSpec((BN, F), lambda i: (i, 0)),
    pl.BlockSpec((2, BN, FE), lambda i: (0, i, 0)),
    pl.BlockSpec((2, BN, FE), lambda i: (0, i, 0)),
    pl.BlockSpec((G, FG), lambda i: (0, 0)),
    pl.BlockSpec((1, 1, BN), lambda i: (i, 0, 0)),
]


def kernel(x, edge_index, edge_attr, u, batch, W1, b1, gamma, beta, W2, b2):
    del b1
    ei3 = edge_index.astype(jnp.int32).reshape(2, NVEC, 128)

    sp, cp = _make_sc_segment()(ei3, edge_attr)

    batch3 = batch.astype(jnp.int32).reshape(NB, 1, BN)
    gammar = jnp.broadcast_to(gamma.reshape(1, H), (8, H))
    betar = jnp.broadcast_to(beta.reshape(1, H), (8, H))
    b2r = jnp.broadcast_to(b2.reshape(1, F), (8, F))

    sh, sh2, outc = pl.pallas_call(
        _moments_body,
        grid=(NB,),
        in_specs=_node_specs + [pl.BlockSpec((IN_DIM, H), lambda i: (0, 0))],
        out_specs=(pl.BlockSpec((8, H), lambda i: (0, 0)),
                   pl.BlockSpec((8, H), lambda i: (0, 0)),
                   pl.BlockSpec((BN, IN_DIM), lambda i: (i, 0))),
        out_shape=(jax.ShapeDtypeStruct((8, H), jnp.float32),
                   jax.ShapeDtypeStruct((8, H), jnp.float32),
                   jax.ShapeDtypeStruct((N, IN_DIM), jnp.float32)),
    )(x, sp, cp, u, batch3, W1)

    W1f, b1f = pl.pallas_call(
        _fold_body,
        out_shape=(jax.ShapeDtypeStruct((IN_DIM, H), jnp.float32),
                   jax.ShapeDtypeStruct((8, H), jnp.float32)),
    )(sh, sh2, gammar, betar, W1)

    y = pl.pallas_call(
        _final_body,
        grid=(NB,),
        in_specs=[
            pl.BlockSpec((BN, IN_DIM), lambda i: (i, 0)),
            pl.BlockSpec((IN_DIM, H), lambda i: (0, 0)),
            pl.BlockSpec((8, H), lambda i: (0, 0)),
            pl.BlockSpec((H, F), lambda i: (0, 0)),
            pl.BlockSpec((8, F), lambda i: (0, 0)),
        ],
        out_specs=pl.BlockSpec((BN, F), lambda i: (i, 0)),
        out_shape=jax.ShapeDtypeStruct((N, F), jnp.float32),
    )(outc, W1f, b1f, W2, b2r)

    return y


# speedup vs baseline: 1.5187x; 1.0187x over previous
import functools

import jax
import jax.numpy as jnp
from jax import lax
from jax.experimental import pallas as pl
from jax.experimental.pallas import tpu as pltpu
from jax.experimental.pallas import tpu_sc as plsc

N = 50000
E = 800000
F = 64
FE = 16
G = 64
H = 256
FG = 16
IN_DIM = F + FE + FG

NP = 50048
ROWS_PER_TILE = NP // 16

NVEC = E // 128
NW = 32
NG_FULL = NVEC // 8
NG_BASE = NG_FULL // NW
NG_EXTRA = NG_FULL - NG_BASE * NW
TAIL_V = NG_FULL * 8

BN = 2000
NB = N // BN


def _sc_body(ei_ref, attr_ref, sums_out, cnts_out,
             sums_sh, cnts_sh, idx_v, attr_v, ones_v, dma_sem, sc_sem):
    cid = lax.axis_index("c")
    sid = lax.axis_index("s")
    w = sid * 2 + cid

    def _fill_z(i, carry):
        attr_v[i, :] = jnp.zeros((16,), jnp.float32)
        return carry
    lax.fori_loop(0, 1024, _fill_z, None)

    def _fill_o(i, carry):
        ones_v[i, :] = jnp.ones((16,), jnp.float32)
        return carry
    lax.fori_loop(0, 128, _fill_o, None)

    base = sid * ROWS_PER_TILE
    for q in range(3):
        pltpu.sync_copy(attr_v, sums_sh.at[pl.ds(base + q * 1024, 1024)])
        pltpu.sync_copy(attr_v, cnts_sh.at[pl.ds(base + q * 1024, 1024)])
    pltpu.sync_copy(attr_v.at[pl.ds(0, 56)],
                    sums_sh.at[pl.ds(base + 3072, 56)])
    pltpu.sync_copy(attr_v.at[pl.ds(0, 56)],
                    cnts_sh.at[pl.ds(base + 3072, 56)])
    plsc.subcore_barrier()

    gbase = w * NG_BASE + jnp.minimum(w, NG_EXTRA)
    ngroups = NG_BASE + jnp.where(w < NG_EXTRA, 1, 0)

    def _group(g2, carry):
        v0 = (gbase + g2) * 8
        fetches = [pltpu.async_copy(ei_ref.at[1, pl.ds(v0, 8)], idx_v,
                                    dma_sem)]
        fetches.append(pltpu.async_copy(
            attr_ref.at[pl.ds(v0 * 128, 8 * 128)], attr_v, dma_sem))
        for c in fetches:
            c.wait()
        scatters = []
        for j in range(8):
            scatters.append(pltpu.async_copy(
                attr_v.at[pl.ds(j * 128, 128)],
                sums_sh.at[idx_v.at[j]], sc_sem, add=True))
            scatters.append(pltpu.async_copy(
                ones_v, cnts_sh.at[idx_v.at[j]], sc_sem, add=True))
        for c in scatters:
            c.wait()
        return carry
    lax.fori_loop(0, ngroups, _group, None)

    @pl.when(w == NW - 1)
    def _tail():
        f1 = pltpu.async_copy(ei_ref.at[1, pl.ds(TAIL_V, 2)],
                              idx_v.at[pl.ds(0, 2)], dma_sem)
        f2 = pltpu.async_copy(attr_ref.at[pl.ds(TAIL_V * 128, 2 * 128)],
                              attr_v.at[pl.ds(0, 2 * 128)], dma_sem)
        f1.wait()
        f2.wait()
        scatters = []
        for j in range(2):
            scatters.append(pltpu.async_copy(
                attr_v.at[pl.ds(j * 128, 128)],
                sums_sh.at[idx_v.at[j]], sc_sem, add=True))
            scatters.append(pltpu.async_copy(
                ones_v, cnts_sh.at[idx_v.at[j]], sc_sem, add=True))
        for c in scatters:
            c.wait()

    plsc.subcore_barrier()

    for acc, dst in ((sums_sh, sums_out), (cnts_sh, cnts_out)):
        for q in range(3):
            pltpu.sync_copy(acc.at[pl.ds(base + q * 1024, 1024)], attr_v)
            pltpu.sync_copy(attr_v, dst.at[cid, pl.ds(base + q * 1024, 1024)])
        pltpu.sync_copy(acc.at[pl.ds(base + 3072, 56)],
                        attr_v.at[pl.ds(0, 56)])
        pltpu.sync_copy(attr_v.at[pl.ds(0, 56)],
                        dst.at[cid, pl.ds(base + 3072, 56)])


@functools.cache
def _make_sc_segment():
    return pl.kernel(
        _sc_body,
        out_type=(jax.ShapeDtypeStruct((2, NP, 16), jnp.float32),
                  jax.ShapeDtypeStruct((2, NP, 16), jnp.float32)),
        mesh=plsc.VectorSubcoreMesh(core_axis_name="c", subcore_axis_name="s",
                                    num_cores=2, num_subcores=16),
        scratch_types=[
            pltpu.VMEM_SHARED((NP, 16), jnp.float32),
            pltpu.VMEM_SHARED((NP, 16), jnp.float32),
            pltpu.VMEM((8, 128), jnp.int32),
            pltpu.VMEM((8 * 128, 16), jnp.float32),
            pltpu.VMEM((128, 16), jnp.float32),
            pltpu.SemaphoreType.DMA,
            pltpu.SemaphoreType.DMA,
        ],
        compiler_params=pltpu.CompilerParams(use_tc_tiling_on_sc=False),
    )


def _build_out(x_ref, sp_ref, cp_ref, u_ref, b_ref):
    x = x_ref[...]
    e = (sp_ref[0] + sp_ref[1]) / jnp.maximum(cp_ref[0] + cp_ref[1], 1.0)
    bidx = b_ref[0, 0, :]
    oh = (bidx[:, None] == lax.broadcasted_iota(jnp.int32, (BN, G), 1)
          ).astype(jnp.float32)
    ub = jnp.dot(oh, u_ref[...], preferred_element_type=jnp.float32)
    return jnp.concatenate([x, e, ub], axis=1)


def _fused_body(x_ref, sp_ref, cp_ref, u_ref, b_ref, W1_ref, gm_ref, bt_ref,
                W2_ref, b2_ref, y_ref, sh_ref, sh2_ref, outc_ref, W1f_ref,
                b1f_ref):
    p = pl.program_id(0)
    i = pl.program_id(1)

    @pl.when((p == 0) & (i == 0))
    def _init():
        sh_ref[...] = jnp.zeros_like(sh_ref)
        sh2_ref[...] = jnp.zeros_like(sh2_ref)

    @pl.when(p == 0)
    def _moments():
        out = _build_out(x_ref, sp_ref, cp_ref, u_ref, b_ref)
        outc_ref[pl.ds(i * BN, BN), :] = out
        g = jnp.dot(out, W1_ref[...], preferred_element_type=jnp.float32)
        sh_ref[...] += jnp.broadcast_to(
            jnp.sum(g, axis=0, keepdims=True), (8, H))
        sh2_ref[...] += jnp.broadcast_to(
            jnp.sum(g * g, axis=0, keepdims=True), (8, H))

    @pl.when((p == 1) & (i == 0))
    def _fold():
        inv_n = jnp.float32(1.0 / N)
        mean_g = sh_ref[0:1, :] * inv_n
        var = sh2_ref[0:1, :] * inv_n - mean_g * mean_g
        scale = gm_ref[0:1, :] * lax.rsqrt(var + 1e-5)
        W1f_ref[...] = W1_ref[...] * scale
        b1f_ref[...] = jnp.broadcast_to(bt_ref[0:1, :] - mean_g * scale,
                                        (8, H))

    @pl.when(p == 1)
    def _final():
        out = outc_ref[pl.ds(i * BN, BN), :]
        g = jnp.dot(out, W1f_ref[...], preferred_element_type=jnp.float32)
        h = jnp.maximum(g + b1f_ref[0:1, :], 0.0)
        y_ref[...] = (jnp.dot(h, W2_ref[...],
                              preferred_element_type=jnp.float32)
                      + b2_ref[0:1, :])


def _node_ix(p, i):
    return jnp.where(p == 0, i, 0)


_node_specs = [
    pl.BlockSpec((BN, F), lambda p, i: (_node_ix(p, i), 0)),
    pl.BlockSpec((2, BN, FE), lambda p, i: (0, _node_ix(p, i), 0)),
    pl.BlockSpec((2, BN, FE), lambda p, i: (0, _node_ix(p, i), 0)),
    pl.BlockSpec((G, FG), lambda p, i: (0, 0)),
    pl.BlockSpec((1, 1, BN), lambda p, i: (_node_ix(p, i), 0, 0)),
]


def kernel(x, edge_index, edge_attr, u, batch, W1, b1, gamma, beta, W2, b2):
    del b1
    ei3 = edge_index.astype(jnp.int32).reshape(2, NVEC, 128)

    sp, cp = _make_sc_segment()(ei3, edge_attr)

    batch3 = batch.astype(jnp.int32).reshape(NB, 1, BN)
    gammar = jnp.broadcast_to(gamma.reshape(1, H), (8, H))
    betar = jnp.broadcast_to(beta.reshape(1, H), (8, H))
    b2r = jnp.broadcast_to(b2.reshape(1, F), (8, F))

    y = pl.pallas_call(
        _fused_body,
        grid=(2, NB),
        in_specs=_node_specs + [
            pl.BlockSpec((IN_DIM, H), lambda p, i: (0, 0)),
            pl.BlockSpec((8, H), lambda p, i: (0, 0)),
            pl.BlockSpec((8, H), lambda p, i: (0, 0)),
            pl.BlockSpec((H, F), lambda p, i: (0, 0)),
            pl.BlockSpec((8, F), lambda p, i: (0, 0)),
        ],
        out_specs=pl.BlockSpec((BN, F), lambda p, i: (i, 0)),
        out_shape=jax.ShapeDtypeStruct((N, F), jnp.float32),
        scratch_shapes=[
            pltpu.VMEM((8, H), jnp.float32),
            pltpu.VMEM((8, H), jnp.float32),
            pltpu.VMEM((N, IN_DIM), jnp.float32),
            pltpu.VMEM((IN_DIM, H), jnp.float32),
            pltpu.VMEM((8, H), jnp.float32),
        ],
    )(x, sp, cp, u, batch3, W1, gammar, betar, W2, b2r)

    return y
